# Initial kernel scaffold; baseline (speedup 1.0000x reference)
import jax
import jax.numpy as jnp
from jax.experimental import pallas as pl


def kernel(inputs, tables, W, b):
    raise NotImplementedError("write your pallas kernel here")


# speedup vs baseline: 2.8308x; 2.8308x over previous
import functools

import jax
import jax.numpy as jnp
from jax import lax
from jax.experimental import pallas as pl
from jax.experimental.pallas import tpu as pltpu
from jax.experimental.pallas import tpu_sc as plsc


def _fuse_tables(tables, W, b):
  NE, V, D = tables.shape
  O = W.shape[1]

  def body(t_ref, w_ref, b_ref, o_ref):
    o_ref[0] = (
        jnp.dot(t_ref[0], w_ref[0], preferred_element_type=jnp.float32)
        + b_ref[...] * (1.0 / NE)
    )

  return pl.pallas_call(
      body,
      grid=(NE,),
      in_specs=[
          pl.BlockSpec((1, V, D), lambda i: (i, 0, 0)),
          pl.BlockSpec((1, D, O), lambda i: (i, 0, 0)),
          pl.BlockSpec((1, O), lambda i: (0, 0)),
      ],
      out_specs=pl.BlockSpec((1, V, O), lambda i: (i, 0, 0)),
      out_shape=jax.ShapeDtypeStruct((NE, V, O), jnp.float32),
  )(tables, W.reshape(NE, D, O), b.reshape(1, O))


def _gather_sum(fused, idx, NC, NS):
  NW, CHUNKS, NE, n = idx.shape
  O = fused.shape[1]
  T = NW * CHUNKS * n
  NV = O // 16
  mesh = plsc.VectorSubcoreMesh(core_axis_name="c", subcore_axis_name="s")

  @functools.partial(
      pl.kernel,
      out_type=jax.ShapeDtypeStruct((T, O), jnp.float32),
      mesh=mesh,
      scratch_types=[
          pltpu.VMEM((NE, n), jnp.int32),
          pltpu.VMEM((n, O), jnp.float32),
          pltpu.VMEM((n, O), jnp.float32),
          pltpu.VMEM((n, O), jnp.float32),
          pltpu.SemaphoreType.DMA,
          pltpu.SemaphoreType.DMA,
      ],
  )
  def k(fused_hbm, idx_hbm, out_hbm, idx_v, acc_v, s0_v, s1_v, semA, semB):
    wid = lax.axis_index("s") * NC + lax.axis_index("c")

    def add_field(stage):
      def row(r, _):
        for kk in range(NV):
          sl = pl.ds(kk * 16, 16)
          plsc.addupdate(acc_v.at[r, sl], stage[r, sl])
        return 0

      lax.fori_loop(0, n, row, 0)

    def zero_acc():
      z = jnp.zeros((16,), jnp.float32)

      def row(r, _):
        for kk in range(NV):
          acc_v[r, pl.ds(kk * 16, 16)] = z
        return 0

      lax.fori_loop(0, n, row, 0)

    def chunk(c, _):
      pltpu.sync_copy(idx_hbm.at[wid, c], idx_v)
      pltpu.async_copy(fused_hbm.at[idx_v.at[0]], s0_v, semA)
      pltpu.async_copy(fused_hbm.at[idx_v.at[1]], s1_v, semB)
      zero_acc()

      def pair(p, _):
        i0 = 2 * p
        pltpu.make_async_copy(fused_hbm.at[idx_v.at[0]], s0_v, semA).wait()
        add_field(s0_v)

        @pl.when(i0 + 2 < NE)
        def _():
          pltpu.async_copy(fused_hbm.at[idx_v.at[i0 + 2]], s0_v, semA)

        pltpu.make_async_copy(fused_hbm.at[idx_v.at[0]], s1_v, semB).wait()
        add_field(s1_v)

        @pl.when(i0 + 3 < NE)
        def _():
          pltpu.async_copy(fused_hbm.at[idx_v.at[i0 + 3]], s1_v, semB)

        return 0

      lax.fori_loop(0, NE // 2, pair, 0)
      pltpu.sync_copy(acc_v, out_hbm.at[pl.ds((wid * CHUNKS + c) * n, n)])
      return 0

    lax.fori_loop(0, CHUNKS, chunk, 0)

  return k(fused, idx)


@jax.jit
def kernel(inputs, tables, W, b):
  NE, V, D = tables.shape
  O = W.shape[1]
  B, L, _ = inputs.shape
  T = B * L

  NC, NS = 2, 16
  NW = NC * NS
  n = 64
  CHUNKS = T // (NW * n)

  fused = _fuse_tables(tables, W, b).reshape(NE * V, O)

  idx = inputs.reshape(NW, CHUNKS, n, NE).transpose(0, 1, 3, 2)
  idx = idx + (jnp.arange(NE, dtype=jnp.int32) * V).reshape(1, 1, NE, 1)

  out = _gather_sum(fused, idx, NC, NS)
  return out.reshape(B, L, O)


# speedup vs baseline: 3.8489x; 1.3597x over previous
import functools

import jax
import jax.numpy as jnp
from jax import lax
from jax.experimental import pallas as pl
from jax.experimental.pallas import tpu as pltpu
from jax.experimental.pallas import tpu_sc as plsc


def _fuse_tables(tables, W, b):
  NE, V, D = tables.shape
  O = W.shape[1]

  def body(t_ref, w_ref, b_ref, o_ref):
    o_ref[0] = (
        jnp.dot(t_ref[0], w_ref[0], preferred_element_type=jnp.float32)
        + b_ref[...] * (1.0 / NE)
    )

  return pl.pallas_call(
      body,
      grid=(NE,),
      in_specs=[
          pl.BlockSpec((1, V, D), lambda i: (i, 0, 0)),
          pl.BlockSpec((1, D, O), lambda i: (i, 0, 0)),
          pl.BlockSpec((1, O), lambda i: (0, 0)),
      ],
      out_specs=pl.BlockSpec((1, V, O), lambda i: (i, 0, 0)),
      out_shape=jax.ShapeDtypeStruct((NE, V, O), jnp.float32),
  )(tables, W.reshape(NE, D, O), b.reshape(1, O))


def _gather_sum(fused, idx, NC, NS, n, G):
  NW, SUPERS, _, HALF = idx.shape
  OW = fused.shape[1]
  O = 2 * OW
  R = 2 * HALF
  NE = R // n
  CHUNKS = SUPERS * G
  T = NW * CHUNKS * n
  mesh = plsc.VectorSubcoreMesh(core_axis_name="c", subcore_axis_name="s")

  @functools.partial(
      pl.kernel,
      out_type=jax.ShapeDtypeStruct((T, O), jnp.float32),
      mesh=mesh,
      scratch_types=[
          pltpu.VMEM((2 * G, HALF), jnp.int32),
          pltpu.VMEM((2, HALF, OW), jnp.int32),
          pltpu.VMEM((2, HALF, OW), jnp.int32),
          pltpu.VMEM((n, O), jnp.float32),
          pltpu.SemaphoreType.DMA,
          pltpu.SemaphoreType.DMA,
          pltpu.SemaphoreType.DMA,
      ],
      compiler_params=pltpu.CompilerParams(needs_layout_passes=False),
  )
  def k(fused_hbm, idx_hbm, out_hbm, idx_v, s0_v, s1_v, outb, semA, semB,
        semO):
    wid = lax.axis_index("s") * NC + lax.axis_index("c")
    base = wid * CHUNKS * n

    def fire(cl, stage, sem):
      for h in range(2):
        pltpu.async_copy(
            fused_hbm.at[idx_v.at[2 * cl + h]], stage.at[h], sem)

    def drain(stage, sem):
      for h in range(2):
        pltpu.make_async_copy(
            fused_hbm.at[idx_v.at[0]], stage.at[h], sem).wait()

    def consume(stage, tok0):
      pltpu.make_async_copy(outb, out_hbm.at[pl.ds(base, n)], semO).wait()

      def g_body(g, _):
        col = 16 * g
        for r in range(n):
          lo_acc = jnp.zeros((16,), jnp.float32)
          hi_acc = jnp.zeros((16,), jnp.float32)
          for i in range(NE):
            j = i * n + r
            vi = stage[j // HALF, j % HALF, pl.ds(col, 16)]
            lo_acc += plsc.bitcast(vi << 16, jnp.float32)
            hi_acc += plsc.bitcast(vi & jnp.int32(-65536), jnp.float32)
          outb[r, pl.ds(2 * col, 16)] = lo_acc
          outb[r, pl.ds(2 * col + 16, 16)] = hi_acc
        return 0

      lax.fori_loop(0, OW // 16, g_body, 0)
      pltpu.async_copy(outb, out_hbm.at[pl.ds(tok0, n)], semO)

    pltpu.async_copy(outb, out_hbm.at[pl.ds(base, n)], semO)

    def super_body(s, _):
      pltpu.sync_copy(idx_hbm.at[wid, s], idx_v)
      c0 = s * G
      fire(0, s0_v, semA)
      fire(1, s1_v, semB)

      def pair(p, _):
        drain(s0_v, semA)
        consume(s0_v, base + (c0 + 2 * p) * n)

        @pl.when(2 * p + 2 < G)
        def _():
          fire(2 * p + 2, s0_v, semA)

        drain(s1_v, semB)
        consume(s1_v, base + (c0 + 2 * p + 1) * n)

        @pl.when(2 * p + 3 < G)
        def _():
          fire(2 * p + 3, s1_v, semB)

        return 0

      lax.fori_loop(0, G // 2, pair, 0)
      return 0

    lax.fori_loop(0, SUPERS, super_body, 0)
    pltpu.make_async_copy(outb, out_hbm.at[pl.ds(base, n)], semO).wait()

  return k(fused, idx)


@jax.jit
def kernel(inputs, tables, W, b):
  NE, V, D = tables.shape
  O = W.shape[1]
  B, L, _ = inputs.shape
  T = B * L

  NC, NS = 2, 16
  NW = NC * NS
  n = 8
  G = 40
  CHUNKS = T // (NW * n)
  SUPERS = CHUNKS // G

  p = (jnp.arange(32) % 2) * 16 + jnp.arange(32) // 2
  Wp = W.reshape(-1, O // 32, 32)[:, :, p].reshape(-1, O)
  bp = b.reshape(O // 32, 32)[:, p].reshape(O)

  fused = _fuse_tables(tables, Wp, bp).reshape(NE * V, O).astype(jnp.bfloat16)
  fused = lax.bitcast_convert_type(
      fused.reshape(NE * V, O // 2, 2), jnp.int32)

  idx = inputs.reshape(NW, CHUNKS, n, NE).transpose(0, 1, 3, 2)
  idx = idx + (jnp.arange(NE, dtype=jnp.int32) * V).reshape(1, 1, NE, 1)
  idx = idx.reshape(NW, SUPERS, 2 * G, NE * n // 2)

  out = _gather_sum(fused, idx, NC, NS, n, G)
  return out.reshape(B, L, O)


# speedup vs baseline: 6.3832x; 1.6584x over previous
import functools

import jax
import jax.numpy as jnp
from jax import lax
from jax.experimental import pallas as pl
from jax.experimental.pallas import tpu as pltpu
from jax.experimental.pallas import tpu_sc as plsc


def _fuse_tables(tables, W, b):
  NE, V, D = tables.shape
  O = W.shape[1]
  OW = O // 2

  def body(t_ref, w_ref, b_ref, o_ref):
    y = (
        jnp.dot(t_ref[0], w_ref[0], preferred_element_type=jnp.float32)
        + b_ref[...] * (1.0 / NE)
    )
    uL = lax.bitcast_convert_type(y[:, :OW], jnp.uint32)
    uH = lax.bitcast_convert_type(y[:, OW:], jnp.uint32)
    half = jnp.uint32(0x7FFF)
    one = jnp.uint32(1)
    rL = (uL + half + ((uL >> 16) & one)) >> 16
    rH = (uH + half + ((uH >> 16) & one)) & jnp.uint32(0xFFFF0000)
    o_ref[0] = lax.bitcast_convert_type(rL | rH, jnp.int32)

  return pl.pallas_call(
      body,
      grid=(NE,),
      in_specs=[
          pl.BlockSpec((1, V, D), lambda i: (i, 0, 0)),
          pl.BlockSpec((1, D, O), lambda i: (i, 0, 0)),
          pl.BlockSpec((1, O), lambda i: (0, 0)),
      ],
      out_specs=pl.BlockSpec((1, V, OW), lambda i: (i, 0, 0)),
      out_shape=jax.ShapeDtypeStruct((NE, V, OW), jnp.int32),
  )(tables, W.reshape(NE, D, O), b.reshape(1, O))


def _gather_sum(fused, idx, pattern, NC, NS, n, G, NE):
  NW, SUPERS, SUP = idx.shape
  OW = fused.shape[1]
  O = 2 * OW
  R = NE * n
  HALF = R // 2
  CHUNKS = SUPERS * G
  T = NW * CHUNKS * n
  mesh = plsc.VectorSubcoreMesh(core_axis_name="c", subcore_axis_name="s")

  @functools.partial(
      pl.kernel,
      out_type=jax.ShapeDtypeStruct((T, O), jnp.float32),
      mesh=mesh,
      scratch_types=[
          pltpu.VMEM((SUP,), jnp.int32),
          pltpu.VMEM((208,), jnp.int32),
          pltpu.VMEM((2, HALF, OW), jnp.int32),
          pltpu.VMEM((2, HALF, OW), jnp.int32),
          pltpu.VMEM((n, O), jnp.float32),
          pltpu.SemaphoreType.DMA,
          pltpu.SemaphoreType.DMA,
          pltpu.SemaphoreType.DMA,
      ],
      compiler_params=pltpu.CompilerParams(needs_layout_passes=False),
  )
  def k(fused_hbm, idx_hbm, pat_hbm, out_hbm, idx_v, pat_v, s0_v, s1_v,
        outb, semA, semB, semO):
    wid = lax.axis_index("s") * NC + lax.axis_index("c")
    base = wid * CHUNKS * n

    pltpu.sync_copy(pat_hbm, pat_v)

    def fire(cl, stage, sem):
      for h in range(2):
        pltpu.async_copy(
            fused_hbm.at[idx_v.at[pl.ds(cl * R + h * HALF, HALF)]],
            stage.at[h], sem)

    def drain(stage, sem):
      for h in range(2):
        pltpu.make_async_copy(
            fused_hbm.at[idx_v.at[pl.ds(0, HALF)]], stage.at[h], sem).wait()

    def consume(stage, tok0):
      pltpu.make_async_copy(outb, out_hbm.at[pl.ds(base, n)], semO).wait()

      def g_body(g, _):
        col = 16 * g
        for r in range(n):
          lo_acc = jnp.zeros((16,), jnp.float32)
          hi_acc = jnp.zeros((16,), jnp.float32)
          for i in range(NE):
            j = r * NE + i
            s = stage[j // HALF, j % HALF, pl.ds(col, 16)]
            lo_acc += plsc.bitcast(s << 16, jnp.float32)
            hi_acc += plsc.bitcast(s & jnp.int32(-65536), jnp.float32)
          outb[r, pl.ds(col, 16)] = lo_acc
          outb[r, pl.ds(OW + col, 16)] = hi_acc
        return 0

      lax.fori_loop(0, OW // 16, g_body, 0)
      pltpu.async_copy(outb, out_hbm.at[pl.ds(tok0, n)], semO)

    pltpu.async_copy(outb, out_hbm.at[pl.ds(base, n)], semO)

    def super_body(s, _):
      pltpu.sync_copy(idx_hbm.at[wid, s], idx_v)

      def adj(q, _):
        for kk in range(13):
          sl = pl.ds(q * R + 16 * kk, 16)
          idx_v[sl] = idx_v[sl] + pat_v[pl.ds(16 * kk, 16)]
        return 0

      lax.fori_loop(0, G, adj, 0)

      c0 = s * G
      fire(0, s0_v, semA)
      fire(1, s1_v, semB)

      def pair(p, _):
        drain(s0_v, semA)
        consume(s0_v, base + (c0 + 2 * p) * n)

        @pl.when(2 * p + 2 < G)
        def _():
          fire(2 * p + 2, s0_v, semA)

        drain(s1_v, semB)
        consume(s1_v, base + (c0 + 2 * p + 1) * n)

        @pl.when(2 * p + 3 < G)
        def _():
          fire(2 * p + 3, s1_v, semB)

        return 0

      lax.fori_loop(0, G // 2, pair, 0)
      return 0

    lax.fori_loop(0, SUPERS, super_body, 0)
    pltpu.make_async_copy(outb, out_hbm.at[pl.ds(base, n)], semO).wait()

  return k(fused, idx, pattern)


@jax.jit
def kernel(inputs, tables, W, b):
  NE, V, D = tables.shape
  O = W.shape[1]
  B, L, _ = inputs.shape
  T = B * L

  NC, NS = 2, 16
  NW = NC * NS
  n = 8
  G = 40
  CHUNKS = T // (NW * n)
  SUPERS = CHUNKS // G

  fused = _fuse_tables(tables, W, b).reshape(NE * V, O // 2)

  idx = inputs.reshape(NW, SUPERS, G * NE * n)
  pattern = jnp.tile(jnp.arange(NE, dtype=jnp.int32) * V, 208 // NE)

  out = _gather_sum(fused, idx, pattern, NC, NS, n, G, NE)
  return out.reshape(B, L, O)
